# baseline (speedup 1.0000x reference)
import functools

import jax
import jax.numpy as jnp
from jax import lax
from jax.experimental import pallas as pl
from jax.experimental.pallas import tpu as pltpu
from jax.experimental.pallas import tpu_sc as plsc

_info = plsc.get_sparse_core_info()
_NC, _NS = _info.num_cores, _info.num_subcores
_NW = _NC * _NS

_K = 8
_CHUNK = _K * 128
_RUN = _CHUNK // 4


def _sc_gather_perm(table, idx_flat, perm):
  n_total = idx_flat.shape[0]
  d = table.shape[1]
  per_w = n_total // _NW
  n_chunks = per_w // _CHUNK

  mesh = plsc.VectorSubcoreMesh(core_axis_name="c", subcore_axis_name="s")

  @functools.partial(
      pl.kernel,
      mesh=mesh,
      out_type=jax.ShapeDtypeStruct((n_total, d), jnp.float32),
      scratch_types=[
          pltpu.VMEM((_CHUNK,), jnp.int32),
          pltpu.VMEM((_CHUNK,), jnp.int32),
          pltpu.VMEM((_K, 128), jnp.int32),
          pltpu.VMEM((_CHUNK, d), jnp.float32),
          pltpu.SemaphoreType.DMA,
      ],
      compiler_params=pltpu.CompilerParams(use_tc_tiling_on_sc=False, needs_layout_passes=False),
  )
  def k(table_hbm, idx_hbm, perm_hbm, out_hbm, idx_s, perm_v, idx_p, rows_v,
        sem):
    wid = lax.axis_index("s") * _NC + lax.axis_index("c")
    g0w = wid * per_w
    pltpu.sync_copy(perm_hbm, perm_v)

    def body(g, carry):
      g0 = pl.multiple_of(g0w + g * _CHUNK, _CHUNK)
      b0 = (g0 // 2048) * 2048
      v0 = (g0 % 2048) // 4
      for q in range(4):
        pltpu.sync_copy(
            idx_hbm.at[pl.ds(pl.multiple_of(b0 + q * 512 + v0, _RUN), _RUN)],
            idx_s.at[pl.ds(q * _RUN, _RUN)],
        )
      for kk in range(_CHUNK // 16):
        vec = plsc.load_gather(idx_s, [perm_v[pl.ds(16 * kk, 16)]])
        idx_p[kk // 8, pl.ds(16 * (kk % 8), 16)] = vec
      copies = []
      for j in range(_K):
        copies.append(
            pltpu.async_copy(
                table_hbm.at[idx_p.at[j]],
                rows_v.at[pl.ds(j * 128, 128), :],
                sem,
            ))
      for c in copies:
        c.wait()
      pltpu.sync_copy(rows_v, out_hbm.at[pl.ds(g0, _CHUNK), :])
      return carry

    lax.fori_loop(0, n_chunks, body, 0)

  return k(table, idx_flat, perm)


def _tc_transform_t(gv3, w4, b128, a, hist, bsz, d):
  rep = 128 // d
  bblk = 2048
  vrows = bblk * d // 128

  def body(g_ref, w_ref, b_ref, a_ref, o_ref):
    e_packed = g_ref[0]
    y = jax.lax.dot_general(
        w_ref[...], e_packed, (((1,), (1,)), ((), ())),
        preferred_element_type=jnp.float32)
    y = y + b_ref[...]
    alpha = a_ref[0]
    y = jnp.where(y >= 0, y, alpha * y)
    for q in range(rep):
      o_ref[0, :, q * vrows:(q + 1) * vrows] = y[q * d:(q + 1) * d, :]

  return pl.pallas_call(
      body,
      grid=(hist, bsz // bblk),
      in_specs=[
          pl.BlockSpec((1, vrows, 128), lambda l, i: (l, i, 0)),
          pl.BlockSpec((128, 128), lambda l, i: (0, 0)),
          pl.BlockSpec((128, 1), lambda l, i: (0, 0)),
          pl.BlockSpec(memory_space=pltpu.SMEM),
      ],
      out_specs=pl.BlockSpec((1, d, bblk), lambda l, i: (l, 0, i)),
      out_shape=jax.ShapeDtypeStruct((hist, d, bsz), jnp.float32),
  )(gv3, w4, b128, a)


def kernel(x, table, W, b, a):
  bsz, hist = x.shape
  d = table.shape[1]
  rep = 128 // d
  n_total = bsz * hist

  idx_flat = x.T.reshape(-1).astype(jnp.int32)
  pp = jnp.arange(_CHUNK, dtype=jnp.int32)
  perm = (pp % 4) * _RUN + pp // 4
  gathered = _sc_gather_perm(table, idx_flat, perm)

  w4 = jnp.kron(jnp.eye(rep, dtype=W.dtype), W)
  b128 = jnp.tile(b, rep).reshape(128, 1)

  gv3 = gathered.reshape(hist, bsz * d // 128, 128)
  out_t = _tc_transform_t(gv3, w4, b128, a, hist, bsz, d)
  return out_t.transpose(2, 0, 1)


# speedup vs baseline: 1.1323x; 1.1323x over previous
import functools

import jax
import jax.numpy as jnp
from jax import lax
from jax.experimental import pallas as pl
from jax.experimental.pallas import tpu as pltpu
from jax.experimental.pallas import tpu_sc as plsc

_info = plsc.get_sparse_core_info()
_NC, _NS = _info.num_cores, _info.num_subcores
_NW = _NC * _NS

_K = 8
_CHUNK = _K * 128
_RUN = _CHUNK // 4
_BB = 4096


def _sc_gather_perm(table, idx2d, perm):
  n_total = idx2d.shape[0] * idx2d.shape[1]
  d = table.shape[1]
  per_w = n_total // _NW
  n_chunks = per_w // _CHUNK

  mesh = plsc.VectorSubcoreMesh(core_axis_name="c", subcore_axis_name="s")

  @functools.partial(
      pl.kernel,
      mesh=mesh,
      out_type=jax.ShapeDtypeStruct((n_total, d), jnp.float32),
      scratch_types=[
          pltpu.VMEM((_CHUNK,), jnp.int32),
          pltpu.VMEM((_CHUNK,), jnp.int32),
          pltpu.VMEM((_K, 128), jnp.int32),
          pltpu.VMEM((_CHUNK, d), jnp.float32),
          pltpu.SemaphoreType.DMA,
      ],
      compiler_params=pltpu.CompilerParams(use_tc_tiling_on_sc=False, needs_layout_passes=False),
  )
  def k(table_hbm, idx_hbm, perm_hbm, out_hbm, idx_s, perm_v, idx_p, rows_v,
        sem):
    wid = lax.axis_index("s") * _NC + lax.axis_index("c")
    g0w = wid * per_w
    bsz = idx_hbm.shape[1]
    pltpu.sync_copy(perm_hbm, perm_v)

    def body(g, carry):
      g0 = pl.multiple_of(g0w + g * _CHUNK, _CHUNK)
      li = g0 // bsz
      bb = g0 % bsz
      b0 = (bb // _BB) * _BB
      v0 = (bb % _BB) // 4
      for q in range(4):
        pltpu.sync_copy(
            idx_hbm.at[li, pl.ds(pl.multiple_of(b0 + q * (_BB // 4) + v0,
                                                _RUN), _RUN)],
            idx_s.at[pl.ds(q * _RUN, _RUN)],
        )
      for kk in range(_CHUNK // 16):
        vec = plsc.load_gather(idx_s, [perm_v[pl.ds(16 * kk, 16)]])
        idx_p[kk // 8, pl.ds(16 * (kk % 8), 16)] = vec
      copies = []
      for j in range(_K):
        copies.append(
            pltpu.async_copy(
                table_hbm.at[idx_p.at[j]],
                rows_v.at[pl.ds(j * 128, 128), :],
                sem,
            ))
      for c in copies:
        c.wait()
      pltpu.sync_copy(rows_v, out_hbm.at[pl.ds(g0, _CHUNK), :])
      return carry

    lax.fori_loop(0, n_chunks, body, 0)

  return k(table, idx2d, perm)


def _tc_transform_t(gv3, w4, b128, a, hist, bsz, d):
  rep = 128 // d
  bblk = _BB
  vrows = bblk * d // 128

  def body(g_ref, w_ref, b_ref, a_ref, o_ref):
    e_packed = g_ref[0]
    y = jax.lax.dot_general(
        w_ref[...], e_packed, (((1,), (1,)), ((), ())),
        preferred_element_type=jnp.float32)
    y = y + b_ref[...]
    alpha = a_ref[0]
    y = jnp.where(y >= 0, y, alpha * y)
    for q in range(rep):
      o_ref[0, :, q * vrows:(q + 1) * vrows] = y[q * d:(q + 1) * d, :]

  return pl.pallas_call(
      body,
      grid=(hist, bsz // bblk),
      in_specs=[
          pl.BlockSpec((1, vrows, 128), lambda l, i: (l, i, 0)),
          pl.BlockSpec((128, 128), lambda l, i: (0, 0)),
          pl.BlockSpec((128, 1), lambda l, i: (0, 0)),
          pl.BlockSpec(memory_space=pltpu.SMEM),
      ],
      out_specs=pl.BlockSpec((1, d, bblk), lambda l, i: (l, 0, i)),
      out_shape=jax.ShapeDtypeStruct((hist, d, bsz), jnp.float32),
  )(gv3, w4, b128, a)


def kernel(x, table, W, b, a):
  bsz, hist = x.shape
  d = table.shape[1]
  rep = 128 // d
  n_total = bsz * hist

  idx2d = x.T.astype(jnp.int32)
  pp = jnp.arange(_CHUNK, dtype=jnp.int32)
  perm = (pp % 4) * _RUN + pp // 4
  gathered = _sc_gather_perm(table, idx2d, perm)

  w4 = jnp.kron(jnp.eye(rep, dtype=W.dtype), W)
  b128 = jnp.tile(b, rep).reshape(128, 1)

  gv3 = gathered.reshape(hist, bsz * d // 128, 128)
  out_t = _tc_transform_t(gv3, w4, b128, a, hist, bsz, d)
  return out_t.transpose(2, 0, 1)
